# Initial kernel scaffold; baseline (speedup 1.0000x reference)
import jax
import jax.numpy as jnp
from jax.experimental import pallas as pl


def kernel(x, edge_index, W1l, b1l, W1r, W2l, b2l, W2r, W3, b3):
    raise NotImplementedError("write your pallas kernel here")


# speedup vs baseline: 2.7102x; 2.7102x over previous
import functools

import jax
import jax.numpy as jnp
from jax import lax
from jax.experimental import pallas as pl
from jax.experimental.pallas import tpu as pltpu
from jax.experimental.pallas import tpu_sc as plsc

_N = 10000
_E = 320000
_D = 128
_B = 128
_NC = 2
_NS = 16
_NW = _NC * _NS
_EROWS = 2560
_EPAD = _EROWS * _B
_RPT = _EROWS // _NW
_NACC = 10240
_ZROWS = _NACC // _NS
_OPT = _N // _NS

_SC_PARAMS = pltpu.CompilerParams(use_tc_tiling_on_sc=False,
                                  needs_layout_passes=False)


def _make_sc_agg(with_counts):
    mesh = plsc.VectorSubcoreMesh(core_axis_name="c", subcore_axis_name="s")

    out_type = [jax.ShapeDtypeStruct((_NC, _N, _D), jnp.float32)]
    scratch = [
        pltpu.VMEM_SHARED((_NACC, _D), jnp.float32),
        pltpu.VMEM((_B, _D), jnp.float32),
        pltpu.VMEM((_B,), jnp.int32),
        pltpu.VMEM((_B,), jnp.int32),
        pltpu.VMEM((16, _D), jnp.float32),
        pltpu.SemaphoreType.DMA,
    ]
    if with_counts:
        out_type.append(jax.ShapeDtypeStruct((_NW * _NACC,), jnp.float32))
        scratch += [
            pltpu.VMEM((_NACC,), jnp.float32),
        ]

    def body(table, src2d, dst2d, *rest):
        if with_counts:
            (sums, cnt_out, accum, rows_v, src_v, dst_v, zbuf, sem,
             cnt_loc) = rest
        else:
            sums, accum, rows_v, src_v, dst_v, zbuf, sem = rest
        c = lax.axis_index("c")
        s = lax.axis_index("s")
        w = c * _NS + s

        def _zrow(r, carry):
            def _zcol(k, carry2):
                zbuf[r, pl.ds(k * 16, 16)] = jnp.zeros((16,), jnp.float32)
                return carry2
            return lax.fori_loop(0, _D // 16, _zcol, carry)
        lax.fori_loop(0, 16, _zrow, 0)

        def _zacc(i, carry):
            pltpu.sync_copy(zbuf, accum.at[pl.ds(s * _ZROWS + i * 16, 16)])
            return carry
        lax.fori_loop(0, _ZROWS // 16, _zacc, 0)

        if with_counts:
            def _zcnt(i, carry):
                cnt_loc[pl.ds(i * 16, 16)] = jnp.zeros((16,), jnp.float32)
                return carry
            lax.fori_loop(0, _NACC // 16, _zcnt, 0)
        plsc.subcore_barrier()

        base = w * _RPT

        def _edge(j, carry):
            pltpu.sync_copy(src2d.at[base + j], src_v)
            gather = pltpu.async_copy(table.at[src_v], rows_v, sem)
            pltpu.sync_copy(dst2d.at[base + j], dst_v)
            if with_counts:
                for k in range(_B // 16):
                    dvec = dst_v[pl.ds(k * 16, 16)]
                    cnts, lastm = plsc.scan_count(dvec)
                    plsc.addupdate_scatter(
                        cnt_loc, [dvec], cnts.astype(jnp.float32), mask=lastm)
            gather.wait()
            pltpu.sync_copy(rows_v, accum.at[dst_v], add=True)
            return carry
        lax.fori_loop(0, _RPT, _edge, 0)

        if with_counts:
            pltpu.sync_copy(cnt_loc, cnt_out.at[pl.ds(w * _NACC, _NACC)])
        plsc.subcore_barrier()

        pltpu.sync_copy(accum.at[pl.ds(s * _OPT, _OPT)],
                        sums.at[c, pl.ds(s * _OPT, _OPT)])

    return pl.kernel(body, out_type=out_type, mesh=mesh,
                     scratch_types=scratch, compiler_params=_SC_PARAMS)


_sc_agg_l1 = _make_sc_agg(True)
_sc_agg_l2 = _make_sc_agg(False)

_BLK = 1000


def _tc_layer1(sums, cnt, x, W1l, b1l, W1r):
    def body(sums_ref, cnt_ref, x_ref, wl_ref, bl_ref, wr_ref, h_ref,
             inv_ref):
        tot = sums_ref[0] + sums_ref[1]
        cntv = jnp.sum(cnt_ref[...], axis=0)
        inv = 1.0 / jnp.maximum(cntv, 1.0)
        mean = tot * inv
        h = jnp.maximum(
            lax.dot_general(mean, wl_ref[...], (((1,), (1,)), ((), ())),
                            preferred_element_type=jnp.float32)
            + bl_ref[...]
            + lax.dot_general(x_ref[...], wr_ref[...], (((1,), (1,)), ((), ())),
                              preferred_element_type=jnp.float32),
            0.0)
        h_ref[...] = h
        inv_ref[...] = inv

    return pl.pallas_call(
        body,
        grid=(_N // _BLK,),
        in_specs=[
            pl.BlockSpec((_NC, _BLK, _D), lambda i: (0, i, 0)),
            pl.BlockSpec((_NW, _BLK, 1), lambda i: (0, i, 0)),
            pl.BlockSpec((_BLK, _D), lambda i: (i, 0)),
            pl.BlockSpec((_D, _D), lambda i: (0, 0)),
            pl.BlockSpec((1, _D), lambda i: (0, 0)),
            pl.BlockSpec((_D, _D), lambda i: (0, 0)),
        ],
        out_specs=[
            pl.BlockSpec((_BLK, _D), lambda i: (i, 0)),
            pl.BlockSpec((_BLK, 1), lambda i: (i, 0)),
        ],
        out_shape=[
            jax.ShapeDtypeStruct((_N, _D), jnp.float32),
            jax.ShapeDtypeStruct((_N, 1), jnp.float32),
        ],
    )(sums, cnt, x, W1l, b1l.reshape(1, _D), W1r)


def _tc_layer2(sums, inv, h1, W2l, b2l, W2r, W3, b3):
    def body(sums_ref, inv_ref, h1_ref, wl_ref, bl_ref, wr_ref, w3_ref,
             b3_ref, score_ref, emb_ref):
        mean = (sums_ref[0] + sums_ref[1]) * inv_ref[...]
        h2 = jnp.maximum(
            lax.dot_general(mean, wl_ref[...], (((1,), (1,)), ((), ())),
                            preferred_element_type=jnp.float32)
            + bl_ref[...]
            + lax.dot_general(h1_ref[...], wr_ref[...], (((1,), (1,)), ((), ())),
                              preferred_element_type=jnp.float32),
            0.0)
        emb_ref[...] = h2
        score_ref[...] = (jnp.sum(h2 * w3_ref[...], axis=1, keepdims=True)
                          + b3_ref[0, 0])

    return pl.pallas_call(
        body,
        grid=(_N // _BLK,),
        in_specs=[
            pl.BlockSpec((_NC, _BLK, _D), lambda i: (0, i, 0)),
            pl.BlockSpec((_BLK, 1), lambda i: (i, 0)),
            pl.BlockSpec((_BLK, _D), lambda i: (i, 0)),
            pl.BlockSpec((_D, _D), lambda i: (0, 0)),
            pl.BlockSpec((1, _D), lambda i: (0, 0)),
            pl.BlockSpec((_D, _D), lambda i: (0, 0)),
            pl.BlockSpec((1, _D), lambda i: (0, 0)),
            pl.BlockSpec(memory_space=pltpu.SMEM),
        ],
        out_specs=[
            pl.BlockSpec((_BLK, 1), lambda i: (i, 0)),
            pl.BlockSpec((_BLK, _D), lambda i: (i, 0)),
        ],
        out_shape=[
            jax.ShapeDtypeStruct((_N, 1), jnp.float32),
            jax.ShapeDtypeStruct((_N, _D), jnp.float32),
        ],
    )(sums, inv, h1, W2l, b2l.reshape(1, _D), W2r, W3, b3.reshape(1, 1))


def kernel(x, edge_index, W1l, b1l, W1r, W2l, b2l, W2r, W3, b3):
    src = edge_index[0].astype(jnp.int32)
    dst = edge_index[1].astype(jnp.int32)
    npad = _EPAD - _E
    src2d = jnp.concatenate(
        [src, jnp.zeros((npad,), jnp.int32)]).reshape(_EROWS, _B)
    dst2d = jnp.concatenate(
        [dst, jnp.full((npad,), _N, jnp.int32)]).reshape(_EROWS, _B)

    sums1, cnt_flat = _sc_agg_l1(x, src2d, dst2d)
    cnt = cnt_flat.reshape(_NW, _NACC, 1)[:, :_N]
    h1, inv = _tc_layer1(sums1, cnt, x, W1l, b1l, W1r)

    [sums2] = _sc_agg_l2(h1, src2d, dst2d)
    score, emb = _tc_layer2(sums2, inv, h1, W2l, b2l, W2r, W3, b3)
    return (score, emb)


# speedup vs baseline: 2.7585x; 1.0178x over previous
import functools

import jax
import jax.numpy as jnp
from jax import lax
from jax.experimental import pallas as pl
from jax.experimental.pallas import tpu as pltpu
from jax.experimental.pallas import tpu_sc as plsc

_N = 10000
_E = 320000
_D = 128
_B = 128
_NC = 2
_NS = 16
_NW = _NC * _NS
_EROWS = 2560
_EPAD = _EROWS * _B
_RPT = _EROWS // _NW
_NACC = 10016
_ZROWS = _NACC // _NS
_OPT = _N // _NS
_K = 2
_IG = 8
_NIG = _RPT // _IG

_SC_PARAMS = pltpu.CompilerParams(use_tc_tiling_on_sc=False,
                                  needs_layout_passes=False)


def _make_sc_agg(with_counts):
    mesh = plsc.VectorSubcoreMesh(core_axis_name="c", subcore_axis_name="s")

    out_type = [jax.ShapeDtypeStruct((_NC, _N, _D), jnp.float32)]
    scratch = [
        pltpu.VMEM_SHARED((_NACC, _D), jnp.float32),
        pltpu.VMEM((_K, _B, _D), jnp.float32),
        pltpu.VMEM((2, _IG, _B), jnp.int32),
        pltpu.VMEM((2, _IG, _B), jnp.int32),
        pltpu.VMEM((16, _D), jnp.float32),
        pltpu.SemaphoreType.DMA((_K,)),
        pltpu.SemaphoreType.DMA((_K,)),
        pltpu.SemaphoreType.DMA((2,)),
        pltpu.SemaphoreType.DMA((2,)),
    ]
    if with_counts:
        out_type.append(jax.ShapeDtypeStruct((_NW * _NACC,), jnp.float32))
        scratch += [
            pltpu.VMEM((_NACC,), jnp.float32),
        ]

    def body(table, src2d, dst2d, *rest):
        if with_counts:
            (sums, cnt_out, accum, rows_v, src_idx, dst_idx, zbuf, gsem,
             ssem, isrc, idst, cnt_loc) = rest
        else:
            (sums, accum, rows_v, src_idx, dst_idx, zbuf, gsem, ssem,
             isrc, idst) = rest
        c = lax.axis_index("c")
        s = lax.axis_index("s")
        w = c * _NS + s
        base = w * _RPT

        pltpu.sync_copy(src2d.at[pl.ds(base, _IG)], src_idx.at[0])
        pltpu.sync_copy(dst2d.at[pl.ds(base, _IG)], dst_idx.at[0])

        def _zrow(r, carry):
            def _zcol(k, carry2):
                zbuf[r, pl.ds(k * 16, 16)] = jnp.zeros((16,), jnp.float32)
                return carry2
            return lax.fori_loop(0, _D // 16, _zcol, carry)
        lax.fori_loop(0, 16, _zrow, 0)

        def _zacc(i, carry):
            pltpu.sync_copy(zbuf, accum.at[pl.ds(s * _ZROWS + i * 16, 16)])
            return carry
        lax.fori_loop(0, _ZROWS // 16, _zacc, 0)
        pltpu.sync_copy(zbuf.at[pl.ds(0, _ZROWS % 16)],
                        accum.at[pl.ds(s * _ZROWS + _ZROWS - _ZROWS % 16,
                                       _ZROWS % 16)])

        if with_counts:
            def _zcnt(i, carry):
                cnt_loc[pl.ds(i * 16, 16)] = jnp.zeros((16,), jnp.float32)
                return carry
            lax.fori_loop(0, _NACC // 16, _zcnt, 0)
        plsc.subcore_barrier()

        pltpu.async_copy(table.at[src_idx.at[0, 0]], rows_v.at[0],
                         gsem.at[0])

        def _edge(j, carry):
            slot = lax.rem(j, _K)
            oslot = lax.rem(j + 1, _K)
            r = lax.rem(j, _IG)
            g = lax.div(j, _IG)
            gb = lax.rem(g, 2)
            ngb = lax.rem(g + 1, 2)

            @pl.when(j >= 1)
            def _():
                pltpu.make_async_copy(
                    table.at[pl.ds(0, _B)], rows_v.at[0],
                    ssem.at[oslot]).wait()

            @pl.when(jnp.logical_and(r == 0, g + 1 < _NIG))
            def _():
                nxt = base + (g + 1) * _IG
                pltpu.async_copy(src2d.at[pl.ds(nxt, _IG)],
                                 src_idx.at[ngb], isrc.at[ngb])
                pltpu.async_copy(dst2d.at[pl.ds(nxt, _IG)],
                                 dst_idx.at[ngb], idst.at[ngb])

            @pl.when(jnp.logical_and(r == _IG - 1, j + 1 < _RPT))
            def _():
                pltpu.make_async_copy(src2d.at[pl.ds(0, _IG)],
                                      src_idx.at[0], isrc.at[ngb]).wait()
                pltpu.make_async_copy(dst2d.at[pl.ds(0, _IG)],
                                      dst_idx.at[0], idst.at[ngb]).wait()

            @pl.when(j + 1 < _RPT)
            def _():
                j1 = j + 1
                r1 = lax.rem(j1, _IG)
                gb1 = lax.rem(lax.div(j1, _IG), 2)
                pltpu.async_copy(table.at[src_idx.at[gb1, r1]],
                                 rows_v.at[oslot], gsem.at[oslot])

            pltpu.make_async_copy(
                table.at[pl.ds(0, _B)], rows_v.at[0], gsem.at[slot]).wait()
            pltpu.async_copy(rows_v.at[slot], accum.at[dst_idx.at[gb, r]],
                             ssem.at[slot], add=True)

            if with_counts:
                for k in range(_B // 16):
                    dvec = dst_idx[gb, r, pl.ds(k * 16, 16)]
                    cnts, lastm = plsc.scan_count(dvec)
                    plsc.addupdate_scatter(
                        cnt_loc, [dvec], cnts.astype(jnp.float32), mask=lastm)
            return carry
        lax.fori_loop(0, _RPT, _edge, 0)
        pltpu.make_async_copy(table.at[pl.ds(0, _B)], rows_v.at[0],
                              ssem.at[(_RPT - 1) % _K]).wait()

        if with_counts:
            pltpu.sync_copy(cnt_loc, cnt_out.at[pl.ds(w * _NACC, _NACC)])
        plsc.subcore_barrier()

        pltpu.sync_copy(accum.at[pl.ds(s * _OPT, _OPT)],
                        sums.at[c, pl.ds(s * _OPT, _OPT)])

    return pl.kernel(body, out_type=out_type, mesh=mesh,
                     scratch_types=scratch, compiler_params=_SC_PARAMS)


_sc_agg_l1 = _make_sc_agg(True)
_sc_agg_l2 = _make_sc_agg(False)

_BLK = 1000


def _tc_layer1(sums, cnt, x, W1l, b1l, W1r):
    def body(sums_ref, cnt_ref, x_ref, wl_ref, bl_ref, wr_ref, h_ref,
             inv_ref):
        tot = sums_ref[0] + sums_ref[1]
        cntv = jnp.sum(cnt_ref[...], axis=0)
        inv = 1.0 / jnp.maximum(cntv, 1.0)
        mean = tot * inv
        h = jnp.maximum(
            lax.dot_general(mean, wl_ref[...], (((1,), (1,)), ((), ())),
                            preferred_element_type=jnp.float32)
            + bl_ref[...]
            + lax.dot_general(x_ref[...], wr_ref[...], (((1,), (1,)), ((), ())),
                              preferred_element_type=jnp.float32),
            0.0)
        h_ref[...] = h
        inv_ref[...] = inv

    return pl.pallas_call(
        body,
        grid=(_N // _BLK,),
        in_specs=[
            pl.BlockSpec((_NC, _BLK, _D), lambda i: (0, i, 0)),
            pl.BlockSpec((_NW, _BLK, 1), lambda i: (0, i, 0)),
            pl.BlockSpec((_BLK, _D), lambda i: (i, 0)),
            pl.BlockSpec((_D, _D), lambda i: (0, 0)),
            pl.BlockSpec((1, _D), lambda i: (0, 0)),
            pl.BlockSpec((_D, _D), lambda i: (0, 0)),
        ],
        out_specs=[
            pl.BlockSpec((_BLK, _D), lambda i: (i, 0)),
            pl.BlockSpec((_BLK, 1), lambda i: (i, 0)),
        ],
        out_shape=[
            jax.ShapeDtypeStruct((_N, _D), jnp.float32),
            jax.ShapeDtypeStruct((_N, 1), jnp.float32),
        ],
    )(sums, cnt, x, W1l, b1l.reshape(1, _D), W1r)


def _tc_layer2(sums, inv, h1, W2l, b2l, W2r, W3, b3):
    def body(sums_ref, inv_ref, h1_ref, wl_ref, bl_ref, wr_ref, w3_ref,
             b3_ref, score_ref, emb_ref):
        mean = (sums_ref[0] + sums_ref[1]) * inv_ref[...]
        h2 = jnp.maximum(
            lax.dot_general(mean, wl_ref[...], (((1,), (1,)), ((), ())),
                            preferred_element_type=jnp.float32)
            + bl_ref[...]
            + lax.dot_general(h1_ref[...], wr_ref[...], (((1,), (1,)), ((), ())),
                              preferred_element_type=jnp.float32),
            0.0)
        emb_ref[...] = h2
        score_ref[...] = (jnp.sum(h2 * w3_ref[...], axis=1, keepdims=True)
                          + b3_ref[0, 0])

    return pl.pallas_call(
        body,
        grid=(_N // _BLK,),
        in_specs=[
            pl.BlockSpec((_NC, _BLK, _D), lambda i: (0, i, 0)),
            pl.BlockSpec((_BLK, 1), lambda i: (i, 0)),
            pl.BlockSpec((_BLK, _D), lambda i: (i, 0)),
            pl.BlockSpec((_D, _D), lambda i: (0, 0)),
            pl.BlockSpec((1, _D), lambda i: (0, 0)),
            pl.BlockSpec((_D, _D), lambda i: (0, 0)),
            pl.BlockSpec((1, _D), lambda i: (0, 0)),
            pl.BlockSpec(memory_space=pltpu.SMEM),
        ],
        out_specs=[
            pl.BlockSpec((_BLK, 1), lambda i: (i, 0)),
            pl.BlockSpec((_BLK, _D), lambda i: (i, 0)),
        ],
        out_shape=[
            jax.ShapeDtypeStruct((_N, 1), jnp.float32),
            jax.ShapeDtypeStruct((_N, _D), jnp.float32),
        ],
    )(sums, inv, h1, W2l, b2l.reshape(1, _D), W2r, W3, b3.reshape(1, 1))


def kernel(x, edge_index, W1l, b1l, W1r, W2l, b2l, W2r, W3, b3):
    src = edge_index[0].astype(jnp.int32)
    dst = edge_index[1].astype(jnp.int32)
    npad = _EPAD - _E
    src2d = jnp.concatenate(
        [src, jnp.zeros((npad,), jnp.int32)]).reshape(_EROWS, _B)
    dst2d = jnp.concatenate(
        [dst, jnp.full((npad,), _N, jnp.int32)]).reshape(_EROWS, _B)

    sums1, cnt_flat = _sc_agg_l1(x, src2d, dst2d)
    cnt = cnt_flat.reshape(_NW, _NACC, 1)[:, :_N]
    h1, inv = _tc_layer1(sums1, cnt, x, W1l, b1l, W1r)

    [sums2] = _sc_agg_l2(h1, src2d, dst2d)
    score, emb = _tc_layer2(sums2, inv, h1, W2l, b2l, W2r, W3, b3)
    return (score, emb)


# speedup vs baseline: 6.5940x; 2.3905x over previous
import functools

import jax
import jax.numpy as jnp
from jax import lax
from jax.experimental import pallas as pl
from jax.experimental.pallas import tpu as pltpu
from jax.experimental.pallas import tpu_sc as plsc

_N = 10000
_E = 320000
_D = 128
_B = 128
_NC = 2
_NS = 16
_NW = _NC * _NS
_EROWS = 2500
_EROWS_PAD = 2504
_RPT = 78
_NACC = 10016
_ZROWS = _NACC // _NS
_OPT = _N // _NS
_K = 2
_IG = 8

_SC_PARAMS = pltpu.CompilerParams(use_tc_tiling_on_sc=False,
                                  needs_layout_passes=False)


def _make_sc_agg(with_counts):
    mesh = plsc.VectorSubcoreMesh(core_axis_name="c", subcore_axis_name="s")

    out_type = [jax.ShapeDtypeStruct((_NC, _N, _D), jnp.float32)]
    scratch = [
        pltpu.VMEM_SHARED((_NACC, _D), jnp.float32),
        pltpu.VMEM((_K, _B, _D), jnp.float32),
        pltpu.VMEM((2, _IG, _B), jnp.int32),
        pltpu.VMEM((2, _IG, _B), jnp.int32),
        pltpu.VMEM((16, _D), jnp.float32),
        pltpu.SemaphoreType.DMA((_K,)),
        pltpu.SemaphoreType.DMA((_K,)),
        pltpu.SemaphoreType.DMA((2,)),
        pltpu.SemaphoreType.DMA((2,)),
    ]
    if with_counts:
        out_type.append(jax.ShapeDtypeStruct((_NW * _NACC,), jnp.float32))
        scratch += [
            pltpu.VMEM((_NACC,), jnp.float32),
        ]

    def body(table, src2d, dst2d, *rest):
        if with_counts:
            (sums, cnt_out, accum, rows_v, src_idx, dst_idx, zbuf, gsem,
             ssem, isrc, idst, cnt_loc) = rest
        else:
            (sums, accum, rows_v, src_idx, dst_idx, zbuf, gsem, ssem,
             isrc, idst) = rest
        c = lax.axis_index("c")
        s = lax.axis_index("s")
        w = c * _NS + s
        base = w * _RPT + jnp.minimum(w, 4)
        nj = _RPT + jnp.where(w < 4, 1, 0)

        pltpu.sync_copy(src2d.at[pl.ds(base, _IG)], src_idx.at[0])
        pltpu.sync_copy(dst2d.at[pl.ds(base, _IG)], dst_idx.at[0])

        def _zrow(r, carry):
            def _zcol(k, carry2):
                zbuf[r, pl.ds(k * 16, 16)] = jnp.zeros((16,), jnp.float32)
                return carry2
            return lax.fori_loop(0, _D // 16, _zcol, carry)
        lax.fori_loop(0, 16, _zrow, 0)

        def _zacc(i, carry):
            pltpu.sync_copy(zbuf, accum.at[pl.ds(s * _ZROWS + i * 16, 16)])
            return carry
        lax.fori_loop(0, _ZROWS // 16, _zacc, 0)
        pltpu.sync_copy(zbuf.at[pl.ds(0, _ZROWS % 16)],
                        accum.at[pl.ds(s * _ZROWS + _ZROWS - _ZROWS % 16,
                                       _ZROWS % 16)])

        if with_counts:
            def _zcnt(i, carry):
                cnt_loc[pl.ds(i * 16, 16)] = jnp.zeros((16,), jnp.float32)
                return carry
            lax.fori_loop(0, _NACC // 16, _zcnt, 0)
        plsc.subcore_barrier()

        pltpu.async_copy(table.at[src_idx.at[0, 0]], rows_v.at[0],
                         gsem.at[0])

        def _edge(j, carry):
            slot = lax.rem(j, _K)
            oslot = lax.rem(j + 1, _K)
            r = lax.rem(j, _IG)
            g = lax.div(j, _IG)
            gb = lax.rem(g, 2)
            ngb = lax.rem(g + 1, 2)

            @pl.when(j >= 1)
            def _():
                pltpu.make_async_copy(
                    table.at[pl.ds(0, _B)], rows_v.at[0],
                    ssem.at[oslot]).wait()

            @pl.when(jnp.logical_and(r == 0, (g + 1) * _IG < nj))
            def _():
                nxt = base + (g + 1) * _IG
                pltpu.async_copy(src2d.at[pl.ds(nxt, _IG)],
                                 src_idx.at[ngb], isrc.at[ngb])
                pltpu.async_copy(dst2d.at[pl.ds(nxt, _IG)],
                                 dst_idx.at[ngb], idst.at[ngb])

            @pl.when(jnp.logical_and(r == _IG - 1, j + 1 < nj))
            def _():
                pltpu.make_async_copy(src2d.at[pl.ds(0, _IG)],
                                      src_idx.at[0], isrc.at[ngb]).wait()
                pltpu.make_async_copy(dst2d.at[pl.ds(0, _IG)],
                                      dst_idx.at[0], idst.at[ngb]).wait()

            @pl.when(j + 1 < nj)
            def _():
                j1 = j + 1
                r1 = lax.rem(j1, _IG)
                gb1 = lax.rem(lax.div(j1, _IG), 2)
                pltpu.async_copy(table.at[src_idx.at[gb1, r1]],
                                 rows_v.at[oslot], gsem.at[oslot])

            pltpu.make_async_copy(
                table.at[pl.ds(0, _B)], rows_v.at[0], gsem.at[slot]).wait()
            pltpu.async_copy(rows_v.at[slot], accum.at[dst_idx.at[gb, r]],
                             ssem.at[slot], add=True)

            if with_counts:
                for k in range(_B // 16):
                    dvec = dst_idx[gb, r, pl.ds(k * 16, 16)]
                    cnts, lastm = plsc.scan_count(dvec)
                    plsc.addupdate_scatter(
                        cnt_loc, [dvec], cnts.astype(jnp.float32), mask=lastm)
            return carry
        lax.fori_loop(0, nj, _edge, 0)
        pltpu.make_async_copy(table.at[pl.ds(0, _B)], rows_v.at[0],
                              ssem.at[lax.rem(nj - 1, _K)]).wait()

        if with_counts:
            pltpu.sync_copy(cnt_loc, cnt_out.at[pl.ds(w * _NACC, _NACC)])
        plsc.subcore_barrier()

        pltpu.sync_copy(accum.at[pl.ds(s * _OPT, _OPT)],
                        sums.at[c, pl.ds(s * _OPT, _OPT)])

    return pl.kernel(body, out_type=out_type, mesh=mesh,
                     scratch_types=scratch, compiler_params=_SC_PARAMS)


_sc_agg_l1 = _make_sc_agg(True)
_sc_agg_l2 = _make_sc_agg(False)

_BLK = 1000


def _tc_layer1(sums, cnt, x, W1l, b1l, W1r):
    def body(sums_ref, cnt_ref, x_ref, wl_ref, bl_ref, wr_ref, h_ref,
             inv_ref):
        tot = sums_ref[0] + sums_ref[1]
        cntv = jnp.sum(cnt_ref[...], axis=0)
        inv = 1.0 / jnp.maximum(cntv, 1.0)
        mean = tot * inv
        h = jnp.maximum(
            lax.dot_general(mean, wl_ref[...], (((1,), (1,)), ((), ())),
                            preferred_element_type=jnp.float32)
            + bl_ref[...]
            + lax.dot_general(x_ref[...], wr_ref[...], (((1,), (1,)), ((), ())),
                              preferred_element_type=jnp.float32),
            0.0)
        h_ref[...] = h
        inv_ref[...] = inv

    return pl.pallas_call(
        body,
        grid=(_N // _BLK,),
        in_specs=[
            pl.BlockSpec((_NC, _BLK, _D), lambda i: (0, i, 0)),
            pl.BlockSpec((_NW, _BLK, 1), lambda i: (0, i, 0)),
            pl.BlockSpec((_BLK, _D), lambda i: (i, 0)),
            pl.BlockSpec((_D, _D), lambda i: (0, 0)),
            pl.BlockSpec((1, _D), lambda i: (0, 0)),
            pl.BlockSpec((_D, _D), lambda i: (0, 0)),
        ],
        out_specs=[
            pl.BlockSpec((_BLK, _D), lambda i: (i, 0)),
            pl.BlockSpec((_BLK, 1), lambda i: (i, 0)),
        ],
        out_shape=[
            jax.ShapeDtypeStruct((_N, _D), jnp.float32),
            jax.ShapeDtypeStruct((_N, 1), jnp.float32),
        ],
    )(sums, cnt, x, W1l, b1l.reshape(1, _D), W1r)


def _tc_layer2(sums, inv, h1, W2l, b2l, W2r, W3, b3):
    def body(sums_ref, inv_ref, h1_ref, wl_ref, bl_ref, wr_ref, w3_ref,
             b3_ref, score_ref, emb_ref):
        mean = (sums_ref[0] + sums_ref[1]) * inv_ref[...]
        h2 = jnp.maximum(
            lax.dot_general(mean, wl_ref[...], (((1,), (1,)), ((), ())),
                            preferred_element_type=jnp.float32)
            + bl_ref[...]
            + lax.dot_general(h1_ref[...], wr_ref[...], (((1,), (1,)), ((), ())),
                              preferred_element_type=jnp.float32),
            0.0)
        emb_ref[...] = h2
        score_ref[...] = (jnp.sum(h2 * w3_ref[...], axis=1, keepdims=True)
                          + b3_ref[0, 0])

    return pl.pallas_call(
        body,
        grid=(_N // _BLK,),
        in_specs=[
            pl.BlockSpec((_NC, _BLK, _D), lambda i: (0, i, 0)),
            pl.BlockSpec((_BLK, 1), lambda i: (i, 0)),
            pl.BlockSpec((_BLK, _D), lambda i: (i, 0)),
            pl.BlockSpec((_D, _D), lambda i: (0, 0)),
            pl.BlockSpec((1, _D), lambda i: (0, 0)),
            pl.BlockSpec((_D, _D), lambda i: (0, 0)),
            pl.BlockSpec((1, _D), lambda i: (0, 0)),
            pl.BlockSpec(memory_space=pltpu.SMEM),
        ],
        out_specs=[
            pl.BlockSpec((_BLK, 1), lambda i: (i, 0)),
            pl.BlockSpec((_BLK, _D), lambda i: (i, 0)),
        ],
        out_shape=[
            jax.ShapeDtypeStruct((_N, 1), jnp.float32),
            jax.ShapeDtypeStruct((_N, _D), jnp.float32),
        ],
    )(sums, inv, h1, W2l, b2l.reshape(1, _D), W2r, W3, b3.reshape(1, 1))


def kernel(x, edge_index, W1l, b1l, W1r, W2l, b2l, W2r, W3, b3):
    src = edge_index[0].astype(jnp.int32)
    dst = edge_index[1].astype(jnp.int32)
    pad = jnp.zeros(((_EROWS_PAD - _EROWS) * _B,), jnp.int32)
    src2d = jnp.concatenate([src, pad]).reshape(_EROWS_PAD, _B)
    dst2d = jnp.concatenate([dst, pad]).reshape(_EROWS_PAD, _B)

    sums1, cnt_flat = _sc_agg_l1(x, src2d, dst2d)
    cnt = cnt_flat.reshape(_NW, _NACC, 1)[:, :_N]
    h1, inv = _tc_layer1(sums1, cnt, x, W1l, b1l, W1r)

    [sums2] = _sc_agg_l2(h1, src2d, dst2d)
    score, emb = _tc_layer2(sums2, inv, h1, W2l, b2l, W2r, W3, b3)
    return (score, emb)


# speedup vs baseline: 13.3567x; 2.0256x over previous
import functools

import jax
import jax.numpy as jnp
from jax import lax
from jax.experimental import pallas as pl
from jax.experimental.pallas import tpu as pltpu
from jax.experimental.pallas import tpu_sc as plsc

_N = 10000
_E = 320000
_D = 128
_B = 128
_NC = 2
_NS = 16
_NW = _NC * _NS
_EROWS = 2500
_EROWS_PAD = 2504
_RPT = 78
_NACC = 10016
_ZROWS = _NACC // _NS
_OPT = _N // _NS
_NP = 10240
_K = 2
_IG = 8

_SC_PARAMS = pltpu.CompilerParams(use_tc_tiling_on_sc=False,
                                  needs_layout_passes=False)


def _make_sc_agg(with_counts):
    mesh = plsc.VectorSubcoreMesh(core_axis_name="c", subcore_axis_name="s")

    out_type = [jax.ShapeDtypeStruct((_NC, _NP, _D), jnp.float32)]
    scratch = [
        pltpu.VMEM_SHARED((_NACC, _D), jnp.float32),
        pltpu.VMEM((_K, _B, _D), jnp.float32),
        pltpu.VMEM((2, _IG, _B), jnp.int32),
        pltpu.VMEM((2, _IG, _B), jnp.int32),
        pltpu.VMEM((16, _D), jnp.float32),
        pltpu.SemaphoreType.DMA((_K,)),
        pltpu.SemaphoreType.DMA((_K,)),
        pltpu.SemaphoreType.DMA((2,)),
        pltpu.SemaphoreType.DMA((2,)),
    ]
    if with_counts:
        out_type.append(jax.ShapeDtypeStruct((_NW * _NP,), jnp.float32))
        scratch += [
            pltpu.VMEM((_NP,), jnp.float32),
        ]

    def body(table, src2d, dst2d, *rest):
        if with_counts:
            (sums, cnt_out, accum, rows_v, src_idx, dst_idx, zbuf, gsem,
             ssem, isrc, idst, cnt_loc) = rest
        else:
            (sums, accum, rows_v, src_idx, dst_idx, zbuf, gsem, ssem,
             isrc, idst) = rest
        c = lax.axis_index("c")
        s = lax.axis_index("s")
        w = c * _NS + s
        base = w * _RPT + jnp.minimum(w, 4)
        nj = _RPT + jnp.where(w < 4, 1, 0)

        pltpu.sync_copy(src2d.at[pl.ds(base, _IG)], src_idx.at[0])
        pltpu.sync_copy(dst2d.at[pl.ds(base, _IG)], dst_idx.at[0])

        def _zrow(r, carry):
            def _zcol(k, carry2):
                zbuf[r, pl.ds(k * 16, 16)] = jnp.zeros((16,), jnp.float32)
                return carry2
            return lax.fori_loop(0, _D // 16, _zcol, carry)
        lax.fori_loop(0, 16, _zrow, 0)

        def _zacc(i, carry):
            pltpu.sync_copy(zbuf, accum.at[pl.ds(s * _ZROWS + i * 16, 16)])
            return carry
        lax.fori_loop(0, _ZROWS // 16, _zacc, 0)
        pltpu.sync_copy(zbuf.at[pl.ds(0, _ZROWS % 16)],
                        accum.at[pl.ds(s * _ZROWS + _ZROWS - _ZROWS % 16,
                                       _ZROWS % 16)])

        if with_counts:
            def _zcnt(i, carry):
                cnt_loc[pl.ds(i * 16, 16)] = jnp.zeros((16,), jnp.float32)
                return carry
            lax.fori_loop(0, _NP // 16, _zcnt, 0)
        plsc.subcore_barrier()

        pltpu.async_copy(table.at[src_idx.at[0, 0]], rows_v.at[0],
                         gsem.at[0])

        def _edge(j, carry):
            slot = lax.rem(j, _K)
            oslot = lax.rem(j + 1, _K)
            r = lax.rem(j, _IG)
            g = lax.div(j, _IG)
            gb = lax.rem(g, 2)
            ngb = lax.rem(g + 1, 2)

            @pl.when(j >= 1)
            def _():
                pltpu.make_async_copy(
                    table.at[pl.ds(0, _B)], rows_v.at[0],
                    ssem.at[oslot]).wait()

            @pl.when(jnp.logical_and(r == 0, (g + 1) * _IG < nj))
            def _():
                nxt = base + (g + 1) * _IG
                pltpu.async_copy(src2d.at[pl.ds(nxt, _IG)],
                                 src_idx.at[ngb], isrc.at[ngb])
                pltpu.async_copy(dst2d.at[pl.ds(nxt, _IG)],
                                 dst_idx.at[ngb], idst.at[ngb])

            @pl.when(jnp.logical_and(r == _IG - 1, j + 1 < nj))
            def _():
                pltpu.make_async_copy(src2d.at[pl.ds(0, _IG)],
                                      src_idx.at[0], isrc.at[ngb]).wait()
                pltpu.make_async_copy(dst2d.at[pl.ds(0, _IG)],
                                      dst_idx.at[0], idst.at[ngb]).wait()

            @pl.when(j + 1 < nj)
            def _():
                j1 = j + 1
                r1 = lax.rem(j1, _IG)
                gb1 = lax.rem(lax.div(j1, _IG), 2)
                pltpu.async_copy(table.at[src_idx.at[gb1, r1]],
                                 rows_v.at[oslot], gsem.at[oslot])

            pltpu.make_async_copy(
                table.at[pl.ds(0, _B)], rows_v.at[0], gsem.at[slot]).wait()
            pltpu.async_copy(rows_v.at[slot], accum.at[dst_idx.at[gb, r]],
                             ssem.at[slot], add=True)

            if with_counts:
                for k in range(_B // 16):
                    dvec = dst_idx[gb, r, pl.ds(k * 16, 16)]
                    cnts, lastm = plsc.scan_count(dvec)
                    plsc.addupdate_scatter(
                        cnt_loc, [dvec], cnts.astype(jnp.float32), mask=lastm)
            return carry
        lax.fori_loop(0, nj, _edge, 0)
        pltpu.make_async_copy(table.at[pl.ds(0, _B)], rows_v.at[0],
                              ssem.at[lax.rem(nj - 1, _K)]).wait()

        if with_counts:
            pltpu.sync_copy(cnt_loc, cnt_out.at[pl.ds(w * _NP, _NP)])
        plsc.subcore_barrier()

        pltpu.sync_copy(accum.at[pl.ds(s * _OPT, _OPT)],
                        sums.at[c, pl.ds(s * _OPT, _OPT)])

        @pl.when(s < _NS - 1)
        def _ztail():
            pltpu.sync_copy(zbuf, sums.at[c, pl.ds(_N + s * 16, 16)])

    return pl.kernel(body, out_type=out_type, mesh=mesh,
                     scratch_types=scratch, compiler_params=_SC_PARAMS)


_sc_agg_l1 = _make_sc_agg(True)
_sc_agg_l2 = _make_sc_agg(False)

_BLK = 1280


def _tc_layer1(sums, cnt, x, W1l, b1l, W1r):
    def body(sums_ref, cnt_ref, x_ref, wl_ref, bl_ref, wr_ref, h_ref,
             inv_ref):
        tot = sums_ref[0] + sums_ref[1]
        cntv = jnp.sum(cnt_ref[...], axis=0)
        inv = (1.0 / jnp.maximum(cntv, 1.0))[:, None]
        mean = tot * inv
        h = jnp.maximum(
            lax.dot_general(mean, wl_ref[...], (((1,), (1,)), ((), ())),
                            preferred_element_type=jnp.float32)
            + bl_ref[...]
            + lax.dot_general(x_ref[...], wr_ref[...], (((1,), (1,)), ((), ())),
                              preferred_element_type=jnp.float32),
            0.0)
        h_ref[...] = h
        inv_ref[...] = inv

    return pl.pallas_call(
        body,
        grid=(_NP // _BLK,),
        in_specs=[
            pl.BlockSpec((_NC, _BLK, _D), lambda i: (0, i, 0)),
            pl.BlockSpec((_NW, _BLK), lambda i: (0, i)),
            pl.BlockSpec((_BLK, _D), lambda i: (i, 0)),
            pl.BlockSpec((_D, _D), lambda i: (0, 0)),
            pl.BlockSpec((1, _D), lambda i: (0, 0)),
            pl.BlockSpec((_D, _D), lambda i: (0, 0)),
        ],
        out_specs=[
            pl.BlockSpec((_BLK, _D), lambda i: (i, 0)),
            pl.BlockSpec((_BLK, 1), lambda i: (i, 0)),
        ],
        out_shape=[
            jax.ShapeDtypeStruct((_NP, _D), jnp.float32),
            jax.ShapeDtypeStruct((_NP, 1), jnp.float32),
        ],
    )(sums, cnt, x, W1l, b1l.reshape(1, _D), W1r)


def _tc_layer2(sums, inv, h1, W2l, b2l, W2r, W3, b3):
    def body(sums_ref, inv_ref, h1_ref, wl_ref, bl_ref, wr_ref, w3_ref,
             b3_ref, score_ref, emb_ref):
        mean = (sums_ref[0] + sums_ref[1]) * inv_ref[...]
        h2 = jnp.maximum(
            lax.dot_general(mean, wl_ref[...], (((1,), (1,)), ((), ())),
                            preferred_element_type=jnp.float32)
            + bl_ref[...]
            + lax.dot_general(h1_ref[...], wr_ref[...], (((1,), (1,)), ((), ())),
                              preferred_element_type=jnp.float32),
            0.0)
        emb_ref[...] = h2
        score_ref[...] = (jnp.sum(h2 * w3_ref[...], axis=1, keepdims=True)
                          + b3_ref[0, 0])

    return pl.pallas_call(
        body,
        grid=(_NP // _BLK,),
        in_specs=[
            pl.BlockSpec((_NC, _BLK, _D), lambda i: (0, i, 0)),
            pl.BlockSpec((_BLK, 1), lambda i: (i, 0)),
            pl.BlockSpec((_BLK, _D), lambda i: (i, 0)),
            pl.BlockSpec((_D, _D), lambda i: (0, 0)),
            pl.BlockSpec((1, _D), lambda i: (0, 0)),
            pl.BlockSpec((_D, _D), lambda i: (0, 0)),
            pl.BlockSpec((1, _D), lambda i: (0, 0)),
            pl.BlockSpec(memory_space=pltpu.SMEM),
        ],
        out_specs=[
            pl.BlockSpec((_BLK, 1), lambda i: (i, 0)),
            pl.BlockSpec((_BLK, _D), lambda i: (i, 0)),
        ],
        out_shape=[
            jax.ShapeDtypeStruct((_NP, 1), jnp.float32),
            jax.ShapeDtypeStruct((_NP, _D), jnp.float32),
        ],
    )(sums, inv, h1, W2l, b2l.reshape(1, _D), W2r, W3, b3.reshape(1, 1))


def kernel(x, edge_index, W1l, b1l, W1r, W2l, b2l, W2r, W3, b3):
    src = edge_index[0].astype(jnp.int32)
    dst = edge_index[1].astype(jnp.int32)
    pad = jnp.zeros(((_EROWS_PAD - _EROWS) * _B,), jnp.int32)
    src2d = jnp.concatenate([src, pad]).reshape(_EROWS_PAD, _B)
    dst2d = jnp.concatenate([dst, pad]).reshape(_EROWS_PAD, _B)

    x_p = jnp.concatenate([x, jnp.zeros((_NP - _N, _D), jnp.float32)])
    sums1, cnt_flat = _sc_agg_l1(x_p, src2d, dst2d)
    cnt = cnt_flat.reshape(_NW, _NP)
    h1, inv = _tc_layer1(sums1, cnt, x_p, W1l, b1l, W1r)

    [sums2] = _sc_agg_l2(h1, src2d, dst2d)
    score, emb = _tc_layer2(sums2, inv, h1, W2l, b2l, W2r, W3, b3)
    return (score[:_N], emb[:_N])


# speedup vs baseline: 13.9674x; 1.0457x over previous
import functools

import jax
import jax.numpy as jnp
from jax import lax
from jax.experimental import pallas as pl
from jax.experimental.pallas import tpu as pltpu
from jax.experimental.pallas import tpu_sc as plsc

_N = 10000
_E = 320000
_D = 128
_B = 128
_NC = 2
_NS = 16
_NW = _NC * _NS
_EROWS = 2500
_EROWS_PAD = 2504
_RPT = 78
_NACC = 10016
_ZROWS = _NACC // _NS
_OPT = _N // _NS
_NP = 10240
_K = 2
_IG = 8

_SC_PARAMS = pltpu.CompilerParams(use_tc_tiling_on_sc=False,
                                  needs_layout_passes=False)


def _make_sc_agg(with_counts):
    mesh = plsc.VectorSubcoreMesh(core_axis_name="c", subcore_axis_name="s")

    out_type = [jax.ShapeDtypeStruct((_NC, _NP, _D), jnp.float32)]
    scratch = [
        pltpu.VMEM_SHARED((_NACC, _D), jnp.float32),
        pltpu.VMEM((_K, _B, _D), jnp.float32),
        pltpu.VMEM((2, _IG, _B), jnp.int32),
        pltpu.VMEM((2, _IG, _B), jnp.int32),
        pltpu.VMEM((16, _D), jnp.float32),
        pltpu.SemaphoreType.DMA((_K,)),
        pltpu.SemaphoreType.DMA((_K,)),
        pltpu.SemaphoreType.DMA((2,)),
        pltpu.SemaphoreType.DMA((2,)),
    ]
    if with_counts:
        out_type.append(jax.ShapeDtypeStruct((_NW * _NP,), jnp.float32))
        scratch += [
            pltpu.VMEM((_NP,), jnp.float32),
        ]

    def body(table, edges, *rest):
        if with_counts:
            (sums, cnt_out, accum, rows_v, src_idx, dst_idx, zbuf, gsem,
             ssem, isrc, idst, cnt_loc) = rest
        else:
            (sums, accum, rows_v, src_idx, dst_idx, zbuf, gsem, ssem,
             isrc, idst) = rest
        c = lax.axis_index("c")
        s = lax.axis_index("s")
        w = c * _NS + s
        base = w * _RPT + jnp.minimum(w, 4)
        nj = _RPT + jnp.where(w < 4, 1, 0)

        pltpu.sync_copy(edges.at[0, pl.ds(base, _IG)], src_idx.at[0])
        pltpu.sync_copy(edges.at[1, pl.ds(base, _IG)], dst_idx.at[0])

        def _zrow(r, carry):
            def _zcol(k, carry2):
                zbuf[r, pl.ds(k * 16, 16)] = jnp.zeros((16,), jnp.float32)
                return carry2
            return lax.fori_loop(0, _D // 16, _zcol, carry)
        lax.fori_loop(0, 16, _zrow, 0)

        def _zacc(i, carry):
            pltpu.sync_copy(zbuf, accum.at[pl.ds(s * _ZROWS + i * 16, 16)])
            return carry
        lax.fori_loop(0, _ZROWS // 16, _zacc, 0)
        pltpu.sync_copy(zbuf.at[pl.ds(0, _ZROWS % 16)],
                        accum.at[pl.ds(s * _ZROWS + _ZROWS - _ZROWS % 16,
                                       _ZROWS % 16)])

        if with_counts:
            def _zcnt(i, carry):
                cnt_loc[pl.ds(i * 16, 16)] = jnp.zeros((16,), jnp.float32)
                return carry
            lax.fori_loop(0, _NP // 16, _zcnt, 0)
        plsc.subcore_barrier()

        pltpu.async_copy(table.at[src_idx.at[0, 0]], rows_v.at[0],
                         gsem.at[0])

        def _edge(j, carry):
            slot = lax.rem(j, _K)
            oslot = lax.rem(j + 1, _K)
            r = lax.rem(j, _IG)
            g = lax.div(j, _IG)
            gb = lax.rem(g, 2)
            ngb = lax.rem(g + 1, 2)

            @pl.when(j >= 1)
            def _():
                pltpu.make_async_copy(
                    table.at[pl.ds(0, _B)], rows_v.at[0],
                    ssem.at[oslot]).wait()

            @pl.when(jnp.logical_and(r == 0, (g + 1) * _IG < nj))
            def _():
                nxt = base + (g + 1) * _IG
                pltpu.async_copy(edges.at[0, pl.ds(nxt, _IG)],
                                 src_idx.at[ngb], isrc.at[ngb])
                pltpu.async_copy(edges.at[1, pl.ds(nxt, _IG)],
                                 dst_idx.at[ngb], idst.at[ngb])

            @pl.when(jnp.logical_and(r == _IG - 1, j + 1 < nj))
            def _():
                pltpu.make_async_copy(edges.at[0, pl.ds(0, _IG)],
                                      src_idx.at[0], isrc.at[ngb]).wait()
                pltpu.make_async_copy(edges.at[1, pl.ds(0, _IG)],
                                      dst_idx.at[0], idst.at[ngb]).wait()

            @pl.when(j + 1 < nj)
            def _():
                j1 = j + 1
                r1 = lax.rem(j1, _IG)
                gb1 = lax.rem(lax.div(j1, _IG), 2)
                pltpu.async_copy(table.at[src_idx.at[gb1, r1]],
                                 rows_v.at[oslot], gsem.at[oslot])

            pltpu.make_async_copy(
                table.at[pl.ds(0, _B)], rows_v.at[0], gsem.at[slot]).wait()
            pltpu.async_copy(rows_v.at[slot], accum.at[dst_idx.at[gb, r]],
                             ssem.at[slot], add=True)

            if with_counts:
                for k in range(_B // 16):
                    dvec = dst_idx[gb, r, pl.ds(k * 16, 16)]
                    cnts, lastm = plsc.scan_count(dvec)
                    plsc.addupdate_scatter(
                        cnt_loc, [dvec], cnts.astype(jnp.float32), mask=lastm)
            return carry
        lax.fori_loop(0, nj, _edge, 0)
        pltpu.make_async_copy(table.at[pl.ds(0, _B)], rows_v.at[0],
                              ssem.at[lax.rem(nj - 1, _K)]).wait()

        if with_counts:
            pltpu.sync_copy(cnt_loc, cnt_out.at[pl.ds(w * _NP, _NP)])
        plsc.subcore_barrier()

        pltpu.sync_copy(accum.at[pl.ds(s * _OPT, _OPT)],
                        sums.at[c, pl.ds(s * _OPT, _OPT)])

        @pl.when(s < _NS - 1)
        def _ztail():
            pltpu.sync_copy(zbuf, sums.at[c, pl.ds(_N + s * 16, 16)])

    return pl.kernel(body, out_type=out_type, mesh=mesh,
                     scratch_types=scratch, compiler_params=_SC_PARAMS)


_sc_agg_l1 = _make_sc_agg(True)
_sc_agg_l2 = _make_sc_agg(False)

_BLK = 1280


def _tc_layer1(sums, cnt, x, W1l, b1l, W1r):
    def body(sums_ref, cnt_ref, x_ref, wl_ref, bl_ref, wr_ref, h_ref,
             inv_ref):
        tot = sums_ref[0] + sums_ref[1]
        cntv = jnp.sum(cnt_ref[...], axis=0)
        inv = (1.0 / jnp.maximum(cntv, 1.0))[:, None]
        mean = tot * inv
        h = jnp.maximum(
            lax.dot_general(mean, wl_ref[...], (((1,), (1,)), ((), ())),
                            preferred_element_type=jnp.float32)
            + bl_ref[...]
            + lax.dot_general(x_ref[...], wr_ref[...], (((1,), (1,)), ((), ())),
                              preferred_element_type=jnp.float32),
            0.0)
        h_ref[...] = h
        inv_ref[...] = inv

    return pl.pallas_call(
        body,
        grid=(_NP // _BLK,),
        in_specs=[
            pl.BlockSpec((_NC, _BLK, _D), lambda i: (0, i, 0)),
            pl.BlockSpec((_NW, _BLK), lambda i: (0, i)),
            pl.BlockSpec((_BLK, _D), lambda i: (i, 0)),
            pl.BlockSpec((_D, _D), lambda i: (0, 0)),
            pl.BlockSpec((1, _D), lambda i: (0, 0)),
            pl.BlockSpec((_D, _D), lambda i: (0, 0)),
        ],
        out_specs=[
            pl.BlockSpec((_BLK, _D), lambda i: (i, 0)),
            pl.BlockSpec((_BLK, 1), lambda i: (i, 0)),
        ],
        out_shape=[
            jax.ShapeDtypeStruct((_NP, _D), jnp.float32),
            jax.ShapeDtypeStruct((_NP, 1), jnp.float32),
        ],
    )(sums, cnt, x, W1l, b1l.reshape(1, _D), W1r)


def _tc_layer2(sums, inv, h1, W2l, b2l, W2r, W3, b3):
    def body(sums_ref, inv_ref, h1_ref, wl_ref, bl_ref, wr_ref, w3_ref,
             b3_ref, score_ref, emb_ref):
        mean = (sums_ref[0] + sums_ref[1]) * inv_ref[...]
        h2 = jnp.maximum(
            lax.dot_general(mean, wl_ref[...], (((1,), (1,)), ((), ())),
                            preferred_element_type=jnp.float32)
            + bl_ref[...]
            + lax.dot_general(h1_ref[...], wr_ref[...], (((1,), (1,)), ((), ())),
                              preferred_element_type=jnp.float32),
            0.0)
        emb_ref[...] = h2
        score_ref[...] = (jnp.sum(h2 * w3_ref[...], axis=1, keepdims=True)
                          + b3_ref[0, 0])

    return pl.pallas_call(
        body,
        grid=(_NP // _BLK,),
        in_specs=[
            pl.BlockSpec((_NC, _BLK, _D), lambda i: (0, i, 0)),
            pl.BlockSpec((_BLK, 1), lambda i: (i, 0)),
            pl.BlockSpec((_BLK, _D), lambda i: (i, 0)),
            pl.BlockSpec((_D, _D), lambda i: (0, 0)),
            pl.BlockSpec((1, _D), lambda i: (0, 0)),
            pl.BlockSpec((_D, _D), lambda i: (0, 0)),
            pl.BlockSpec((1, _D), lambda i: (0, 0)),
            pl.BlockSpec(memory_space=pltpu.SMEM),
        ],
        out_specs=[
            pl.BlockSpec((_BLK, 1), lambda i: (i, 0)),
            pl.BlockSpec((_BLK, _D), lambda i: (i, 0)),
        ],
        out_shape=[
            jax.ShapeDtypeStruct((_NP, 1), jnp.float32),
            jax.ShapeDtypeStruct((_NP, _D), jnp.float32),
        ],
    )(sums, inv, h1, W2l, b2l.reshape(1, _D), W2r, W3, b3.reshape(1, 1))


def kernel(x, edge_index, W1l, b1l, W1r, W2l, b2l, W2r, W3, b3):
    ei = edge_index.astype(jnp.int32)
    edges = jnp.pad(ei, ((0, 0), (0, (_EROWS_PAD - _EROWS) * _B)))
    edges = edges.reshape(2, _EROWS_PAD, _B)

    x_p = jnp.concatenate([x, jnp.zeros((_NP - _N, _D), jnp.float32)])
    sums1, cnt_flat = _sc_agg_l1(x_p, edges)
    cnt = cnt_flat.reshape(_NW, _NP)
    h1, inv = _tc_layer1(sums1, cnt, x_p, W1l, b1l, W1r)

    [sums2] = _sc_agg_l2(h1, edges)
    score, emb = _tc_layer2(sums2, inv, h1, W2l, b2l, W2r, W3, b3)
    return (score[:_N], emb[:_N])


# speedup vs baseline: 14.1298x; 1.0116x over previous
import functools

import jax
import jax.numpy as jnp
from jax import lax
from jax.experimental import pallas as pl
from jax.experimental.pallas import tpu as pltpu
from jax.experimental.pallas import tpu_sc as plsc

_N = 10000
_E = 320000
_D = 128
_B = 128
_NC = 2
_NS = 16
_NW = _NC * _NS
_EROWS = 2500
_EROWS_PAD = 2504
_RPT = 78
_NACC = 10016
_ZROWS = _NACC // _NS
_OPT = _N // _NS
_NP = 10240
_K = 2
_IG = 8

_SC_PARAMS = pltpu.CompilerParams(use_tc_tiling_on_sc=False,
                                  needs_layout_passes=False)


def _make_sc_agg(with_counts):
    mesh = plsc.VectorSubcoreMesh(core_axis_name="c", subcore_axis_name="s")

    out_type = [jax.ShapeDtypeStruct((_NC, _NP, _D), jnp.float32)]
    scratch = [
        pltpu.VMEM_SHARED((_NACC, _D), jnp.float32),
        pltpu.VMEM((_K, _B, _D), jnp.float32),
        pltpu.VMEM((2, _IG, _B), jnp.int32),
        pltpu.VMEM((2, _IG, _B), jnp.int32),
        pltpu.VMEM((16, _D), jnp.float32),
        pltpu.SemaphoreType.DMA((_K,)),
        pltpu.SemaphoreType.DMA((_K,)),
        pltpu.SemaphoreType.DMA((2,)),
        pltpu.SemaphoreType.DMA((2,)),
    ]
    if with_counts:
        out_type.append(jax.ShapeDtypeStruct((_NW * _NP,), jnp.float32))
        scratch += [
            pltpu.VMEM((_NP,), jnp.float32),
        ]

    def body(table, edges, *rest):
        if with_counts:
            (sums, cnt_out, accum, rows_v, src_idx, dst_idx, zbuf, gsem,
             ssem, isrc, idst, cnt_loc) = rest
        else:
            (sums, accum, rows_v, src_idx, dst_idx, zbuf, gsem, ssem,
             isrc, idst) = rest
        c = lax.axis_index("c")
        s = lax.axis_index("s")
        w = c * _NS + s
        base = w * _RPT + jnp.minimum(w, 4)
        nj = _RPT + jnp.where(w < 4, 1, 0)

        pltpu.sync_copy(edges.at[0, pl.ds(base, _IG)], src_idx.at[0])
        pltpu.sync_copy(edges.at[1, pl.ds(base, _IG)], dst_idx.at[0])

        def _zrow(r, carry):
            def _zcol(k, carry2):
                zbuf[r, pl.ds(k * 16, 16)] = jnp.zeros((16,), jnp.float32)
                return carry2
            return lax.fori_loop(0, _D // 16, _zcol, carry)
        lax.fori_loop(0, 16, _zrow, 0)

        def _zacc(i, carry):
            pltpu.sync_copy(zbuf, accum.at[pl.ds(s * _ZROWS + i * 16, 16)])
            return carry
        lax.fori_loop(0, _ZROWS // 16, _zacc, 0)
        pltpu.sync_copy(zbuf.at[pl.ds(0, _ZROWS % 16)],
                        accum.at[pl.ds(s * _ZROWS + _ZROWS - _ZROWS % 16,
                                       _ZROWS % 16)])

        if with_counts:
            def _zcnt(i, carry):
                cnt_loc[pl.ds(i * 16, 16)] = jnp.zeros((16,), jnp.float32)
                return carry
            lax.fori_loop(0, _NP // 16, _zcnt, 0)
        plsc.subcore_barrier()

        pltpu.async_copy(table.at[src_idx.at[0, 0]], rows_v.at[0],
                         gsem.at[0])

        def _edge(j, carry):
            slot = lax.rem(j, _K)
            oslot = lax.rem(j + 1, _K)
            r = lax.rem(j, _IG)
            g = lax.div(j, _IG)
            gb = lax.rem(g, 2)
            ngb = lax.rem(g + 1, 2)

            @pl.when(j >= 1)
            def _():
                pltpu.make_async_copy(
                    table.at[pl.ds(0, _B)], rows_v.at[0],
                    ssem.at[oslot]).wait()

            @pl.when(jnp.logical_and(r == 0, (g + 1) * _IG < nj))
            def _():
                nxt = base + (g + 1) * _IG
                pltpu.async_copy(edges.at[0, pl.ds(nxt, _IG)],
                                 src_idx.at[ngb], isrc.at[ngb])
                pltpu.async_copy(edges.at[1, pl.ds(nxt, _IG)],
                                 dst_idx.at[ngb], idst.at[ngb])

            @pl.when(jnp.logical_and(r == _IG - 1, j + 1 < nj))
            def _():
                pltpu.make_async_copy(edges.at[0, pl.ds(0, _IG)],
                                      src_idx.at[0], isrc.at[ngb]).wait()
                pltpu.make_async_copy(edges.at[1, pl.ds(0, _IG)],
                                      dst_idx.at[0], idst.at[ngb]).wait()

            @pl.when(j + 1 < nj)
            def _():
                j1 = j + 1
                r1 = lax.rem(j1, _IG)
                gb1 = lax.rem(lax.div(j1, _IG), 2)
                pltpu.async_copy(table.at[src_idx.at[gb1, r1]],
                                 rows_v.at[oslot], gsem.at[oslot])

            pltpu.make_async_copy(
                table.at[pl.ds(0, _B)], rows_v.at[0], gsem.at[slot]).wait()
            pltpu.async_copy(rows_v.at[slot], accum.at[dst_idx.at[gb, r]],
                             ssem.at[slot], add=True)

            if with_counts:
                for k in range(_B // 16):
                    dvec = dst_idx[gb, r, pl.ds(k * 16, 16)]
                    cnts, lastm = plsc.scan_count(dvec)
                    plsc.addupdate_scatter(
                        cnt_loc, [dvec], cnts.astype(jnp.float32), mask=lastm)
            return carry
        lax.fori_loop(0, nj, _edge, 0)
        pltpu.make_async_copy(table.at[pl.ds(0, _B)], rows_v.at[0],
                              ssem.at[lax.rem(nj - 1, _K)]).wait()

        if with_counts:
            pltpu.sync_copy(cnt_loc, cnt_out.at[pl.ds(w * _NP, _NP)])
        plsc.subcore_barrier()

        pltpu.sync_copy(accum.at[pl.ds(s * _OPT, _OPT)],
                        sums.at[c, pl.ds(s * _OPT, _OPT)])

        @pl.when(s < _NS - 1)
        def _ztail():
            pltpu.sync_copy(zbuf, sums.at[c, pl.ds(_N + s * 16, 16)])

    return pl.kernel(body, out_type=out_type, mesh=mesh,
                     scratch_types=scratch, compiler_params=_SC_PARAMS)


_sc_agg_l1 = _make_sc_agg(True)
_sc_agg_l2 = _make_sc_agg(False)

_BLK = 1280


def _tc_pre(x, W, b):
    def body(x_ref, w_ref, b_ref, xr_ref):
        xr_ref[...] = (
            lax.dot_general(x_ref[...], w_ref[...], (((1,), (1,)), ((), ())),
                            preferred_element_type=jnp.float32)
            + b_ref[...])

    return pl.pallas_call(
        body,
        grid=(_NP // _BLK,),
        in_specs=[
            pl.BlockSpec((_BLK, _D), lambda i: (i, 0)),
            pl.BlockSpec((_D, _D), lambda i: (0, 0)),
            pl.BlockSpec((1, _D), lambda i: (0, 0)),
        ],
        out_specs=pl.BlockSpec((_BLK, _D), lambda i: (i, 0)),
        out_shape=jax.ShapeDtypeStruct((_NP, _D), jnp.float32),
    )(x, W, b.reshape(1, _D))


def _tc_post1(sums, cnt, xr, W1l):
    def body(sums_ref, cnt_ref, xr_ref, wl_ref, h_ref, inv_ref):
        tot = sums_ref[0] + sums_ref[1]
        cntv = jnp.sum(cnt_ref[...], axis=0)
        inv = (1.0 / jnp.maximum(cntv, 1.0))[:, None]
        mean = tot * inv
        h = jnp.maximum(
            lax.dot_general(mean, wl_ref[...], (((1,), (1,)), ((), ())),
                            preferred_element_type=jnp.float32)
            + xr_ref[...],
            0.0)
        h_ref[...] = h
        inv_ref[...] = inv

    return pl.pallas_call(
        body,
        grid=(_NP // _BLK,),
        in_specs=[
            pl.BlockSpec((_NC, _BLK, _D), lambda i: (0, i, 0)),
            pl.BlockSpec((_NW, _BLK), lambda i: (0, i)),
            pl.BlockSpec((_BLK, _D), lambda i: (i, 0)),
            pl.BlockSpec((_D, _D), lambda i: (0, 0)),
        ],
        out_specs=[
            pl.BlockSpec((_BLK, _D), lambda i: (i, 0)),
            pl.BlockSpec((_BLK, 1), lambda i: (i, 0)),
        ],
        out_shape=[
            jax.ShapeDtypeStruct((_NP, _D), jnp.float32),
            jax.ShapeDtypeStruct((_NP, 1), jnp.float32),
        ],
    )(sums, cnt, xr, W1l)


def _tc_post2(sums, inv, xr, W2l, W3, b3):
    def body(sums_ref, inv_ref, xr_ref, wl_ref, w3_ref, b3_ref,
             score_ref, emb_ref):
        mean = (sums_ref[0] + sums_ref[1]) * inv_ref[...]
        h2 = jnp.maximum(
            lax.dot_general(mean, wl_ref[...], (((1,), (1,)), ((), ())),
                            preferred_element_type=jnp.float32)
            + xr_ref[...],
            0.0)
        emb_ref[...] = h2
        score_ref[...] = (jnp.sum(h2 * w3_ref[...], axis=1, keepdims=True)
                          + b3_ref[0, 0])

    return pl.pallas_call(
        body,
        grid=(_NP // _BLK,),
        in_specs=[
            pl.BlockSpec((_NC, _BLK, _D), lambda i: (0, i, 0)),
            pl.BlockSpec((_BLK, 1), lambda i: (i, 0)),
            pl.BlockSpec((_BLK, _D), lambda i: (i, 0)),
            pl.BlockSpec((_D, _D), lambda i: (0, 0)),
            pl.BlockSpec((1, _D), lambda i: (0, 0)),
            pl.BlockSpec(memory_space=pltpu.SMEM),
        ],
        out_specs=[
            pl.BlockSpec((_BLK, 1), lambda i: (i, 0)),
            pl.BlockSpec((_BLK, _D), lambda i: (i, 0)),
        ],
        out_shape=[
            jax.ShapeDtypeStruct((_N, 1), jnp.float32),
            jax.ShapeDtypeStruct((_N, _D), jnp.float32),
        ],
    )(sums, inv, xr, W2l, W3, b3.reshape(1, 1))


def kernel(x, edge_index, W1l, b1l, W1r, W2l, b2l, W2r, W3, b3):
    ei = edge_index.astype(jnp.int32)
    edges = jnp.pad(ei, ((0, 0), (0, (_EROWS_PAD - _EROWS) * _B)))
    edges = edges.reshape(2, _EROWS_PAD, _B)

    x_p = jnp.concatenate([x, jnp.zeros((_NP - _N, _D), jnp.float32)])
    sums1, cnt_flat = _sc_agg_l1(x_p, edges)
    xr1 = _tc_pre(x_p, W1r, b1l)
    cnt = cnt_flat.reshape(_NW, _NP)
    h1, inv = _tc_post1(sums1, cnt, xr1, W1l)

    [sums2] = _sc_agg_l2(h1, edges)
    xr2 = _tc_pre(h1, W2r, b2l)
    score, emb = _tc_post2(sums2, inv, xr2, W2l, W3, b3)
    return (score, emb)


# speedup vs baseline: 14.1406x; 1.0008x over previous
import functools

import jax
import jax.numpy as jnp
from jax import lax
from jax.experimental import pallas as pl
from jax.experimental.pallas import tpu as pltpu
from jax.experimental.pallas import tpu_sc as plsc

_N = 10000
_E = 320000
_D = 128
_B = 128
_NC = 2
_NS = 16
_NW = _NC * _NS
_EROWS = 2500
_EROWS_PAD = 2504
_RPT = 78
_NACC = 10016
_ZROWS = _NACC // _NS
_OPT = _N // _NS
_NP = 10240
_K = 2
_IG = 8

_SC_PARAMS = pltpu.CompilerParams(use_tc_tiling_on_sc=False,
                                  needs_layout_passes=False)


def _make_sc_agg(with_counts):
    mesh = plsc.VectorSubcoreMesh(core_axis_name="c", subcore_axis_name="s")

    out_type = [jax.ShapeDtypeStruct((_NC, _NP, _D), jnp.float32)]
    scratch = [
        pltpu.VMEM_SHARED((_NACC, _D), jnp.float32),
        pltpu.VMEM((_K, _B, _D), jnp.float32),
        pltpu.VMEM((2, _IG, _B), jnp.int32),
        pltpu.VMEM((2, _IG, _B), jnp.int32),
        pltpu.VMEM((16, _D), jnp.float32),
        pltpu.SemaphoreType.DMA((_K,)),
        pltpu.SemaphoreType.DMA((_K,)),
        pltpu.SemaphoreType.DMA((2,)),
        pltpu.SemaphoreType.DMA((2,)),
    ]
    if with_counts:
        out_type.append(jax.ShapeDtypeStruct((_NW, _NP), jnp.float32))
        scratch += [
            pltpu.VMEM((_NP,), jnp.float32),
        ]

    def body(table, edges, *rest):
        if with_counts:
            (sums, cnt_out, accum, rows_v, src_idx, dst_idx, zbuf, gsem,
             ssem, isrc, idst, cnt_loc) = rest
        else:
            (sums, accum, rows_v, src_idx, dst_idx, zbuf, gsem, ssem,
             isrc, idst) = rest
        c = lax.axis_index("c")
        s = lax.axis_index("s")
        w = c * _NS + s
        base = w * _RPT + jnp.minimum(w, 4)
        nj = _RPT + jnp.where(w < 4, 1, 0)

        pltpu.sync_copy(edges.at[0, pl.ds(base, _IG)], src_idx.at[0])
        pltpu.sync_copy(edges.at[1, pl.ds(base, _IG)], dst_idx.at[0])

        def _zrow(r, carry):
            def _zcol(k, carry2):
                zbuf[r, pl.ds(k * 16, 16)] = jnp.zeros((16,), jnp.float32)
                return carry2
            return lax.fori_loop(0, _D // 16, _zcol, carry)
        lax.fori_loop(0, 16, _zrow, 0)

        def _zacc(i, carry):
            pltpu.sync_copy(zbuf, accum.at[pl.ds(s * _ZROWS + i * 16, 16)])
            return carry
        lax.fori_loop(0, _ZROWS // 16, _zacc, 0)
        pltpu.sync_copy(zbuf.at[pl.ds(0, _ZROWS % 16)],
                        accum.at[pl.ds(s * _ZROWS + _ZROWS - _ZROWS % 16,
                                       _ZROWS % 16)])

        if with_counts:
            def _zcnt(i, carry):
                cnt_loc[pl.ds(i * 16, 16)] = jnp.zeros((16,), jnp.float32)
                return carry
            lax.fori_loop(0, _NP // 16, _zcnt, 0)
        plsc.subcore_barrier()

        pltpu.async_copy(table.at[src_idx.at[0, 0]], rows_v.at[0],
                         gsem.at[0])

        def _edge(j, carry):
            slot = lax.rem(j, _K)
            oslot = lax.rem(j + 1, _K)
            r = lax.rem(j, _IG)
            g = lax.div(j, _IG)
            gb = lax.rem(g, 2)
            ngb = lax.rem(g + 1, 2)

            @pl.when(j >= 1)
            def _():
                pltpu.make_async_copy(
                    table.at[pl.ds(0, _B)], rows_v.at[0],
                    ssem.at[oslot]).wait()

            @pl.when(jnp.logical_and(r == 0, (g + 1) * _IG < nj))
            def _():
                nxt = base + (g + 1) * _IG
                pltpu.async_copy(edges.at[0, pl.ds(nxt, _IG)],
                                 src_idx.at[ngb], isrc.at[ngb])
                pltpu.async_copy(edges.at[1, pl.ds(nxt, _IG)],
                                 dst_idx.at[ngb], idst.at[ngb])

            @pl.when(jnp.logical_and(r == _IG - 1, j + 1 < nj))
            def _():
                pltpu.make_async_copy(edges.at[0, pl.ds(0, _IG)],
                                      src_idx.at[0], isrc.at[ngb]).wait()
                pltpu.make_async_copy(edges.at[1, pl.ds(0, _IG)],
                                      dst_idx.at[0], idst.at[ngb]).wait()

            @pl.when(j + 1 < nj)
            def _():
                j1 = j + 1
                r1 = lax.rem(j1, _IG)
                gb1 = lax.rem(lax.div(j1, _IG), 2)
                pltpu.async_copy(table.at[src_idx.at[gb1, r1]],
                                 rows_v.at[oslot], gsem.at[oslot])

            pltpu.make_async_copy(
                table.at[pl.ds(0, _B)], rows_v.at[0], gsem.at[slot]).wait()
            pltpu.async_copy(rows_v.at[slot], accum.at[dst_idx.at[gb, r]],
                             ssem.at[slot], add=True)

            if with_counts:
                for k in range(_B // 16):
                    dvec = dst_idx[gb, r, pl.ds(k * 16, 16)]
                    cnts, lastm = plsc.scan_count(dvec)
                    plsc.addupdate_scatter(
                        cnt_loc, [dvec], cnts.astype(jnp.float32), mask=lastm)
            return carry
        lax.fori_loop(0, nj, _edge, 0)
        pltpu.make_async_copy(table.at[pl.ds(0, _B)], rows_v.at[0],
                              ssem.at[lax.rem(nj - 1, _K)]).wait()

        if with_counts:
            pltpu.sync_copy(cnt_loc, cnt_out.at[w])
        plsc.subcore_barrier()

        pltpu.sync_copy(accum.at[pl.ds(s * _OPT, _OPT)],
                        sums.at[c, pl.ds(s * _OPT, _OPT)])

        @pl.when(s < _NS - 1)
        def _ztail():
            pltpu.sync_copy(zbuf, sums.at[c, pl.ds(_N + s * 16, 16)])

    return pl.kernel(body, out_type=out_type, mesh=mesh,
                     scratch_types=scratch, compiler_params=_SC_PARAMS)


_sc_agg_l1 = _make_sc_agg(True)
_sc_agg_l2 = _make_sc_agg(False)

_BLK = 1280


def _tc_pre(x, W, b):
    def body(x_ref, w_ref, b_ref, xr_ref):
        xr_ref[...] = (
            lax.dot_general(x_ref[...], w_ref[...], (((1,), (1,)), ((), ())),
                            preferred_element_type=jnp.float32)
            + b_ref[...])

    return pl.pallas_call(
        body,
        grid=(_NP // _BLK,),
        in_specs=[
            pl.BlockSpec((_BLK, _D), lambda i: (i, 0)),
            pl.BlockSpec((_D, _D), lambda i: (0, 0)),
            pl.BlockSpec((1, _D), lambda i: (0, 0)),
        ],
        out_specs=pl.BlockSpec((_BLK, _D), lambda i: (i, 0)),
        out_shape=jax.ShapeDtypeStruct((_NP, _D), jnp.float32),
    )(x, W, b.reshape(1, _D))


def _tc_post1(sums, cnt, xr, W1l):
    def body(sums_ref, cnt_ref, xr_ref, wl_ref, h_ref, inv_ref):
        tot = sums_ref[0] + sums_ref[1]
        cntv = jnp.sum(cnt_ref[...], axis=0)
        inv = (1.0 / jnp.maximum(cntv, 1.0))[:, None]
        mean = tot * inv
        h = jnp.maximum(
            lax.dot_general(mean, wl_ref[...], (((1,), (1,)), ((), ())),
                            preferred_element_type=jnp.float32)
            + xr_ref[...],
            0.0)
        h_ref[...] = h
        inv_ref[...] = inv

    return pl.pallas_call(
        body,
        grid=(_NP // _BLK,),
        in_specs=[
            pl.BlockSpec((_NC, _BLK, _D), lambda i: (0, i, 0)),
            pl.BlockSpec((_NW, _BLK), lambda i: (0, i)),
            pl.BlockSpec((_BLK, _D), lambda i: (i, 0)),
            pl.BlockSpec((_D, _D), lambda i: (0, 0)),
        ],
        out_specs=[
            pl.BlockSpec((_BLK, _D), lambda i: (i, 0)),
            pl.BlockSpec((_BLK, 1), lambda i: (i, 0)),
        ],
        out_shape=[
            jax.ShapeDtypeStruct((_NP, _D), jnp.float32),
            jax.ShapeDtypeStruct((_NP, 1), jnp.float32),
        ],
    )(sums, cnt, xr, W1l)


def _tc_post2(sums, inv, xr, W2l, W3, b3):
    def body(sums_ref, inv_ref, xr_ref, wl_ref, w3_ref, b3_ref,
             score_ref, emb_ref):
        mean = (sums_ref[0] + sums_ref[1]) * inv_ref[...]
        h2 = jnp.maximum(
            lax.dot_general(mean, wl_ref[...], (((1,), (1,)), ((), ())),
                            preferred_element_type=jnp.float32)
            + xr_ref[...],
            0.0)
        emb_ref[...] = h2
        score_ref[...] = (jnp.sum(h2 * w3_ref[...], axis=1, keepdims=True)
                          + b3_ref[0, 0])

    return pl.pallas_call(
        body,
        grid=(_NP // _BLK,),
        in_specs=[
            pl.BlockSpec((_NC, _BLK, _D), lambda i: (0, i, 0)),
            pl.BlockSpec((_BLK, 1), lambda i: (i, 0)),
            pl.BlockSpec((_BLK, _D), lambda i: (i, 0)),
            pl.BlockSpec((_D, _D), lambda i: (0, 0)),
            pl.BlockSpec((1, _D), lambda i: (0, 0)),
            pl.BlockSpec(memory_space=pltpu.SMEM),
        ],
        out_specs=[
            pl.BlockSpec((_BLK, 1), lambda i: (i, 0)),
            pl.BlockSpec((_BLK, _D), lambda i: (i, 0)),
        ],
        out_shape=[
            jax.ShapeDtypeStruct((_N, 1), jnp.float32),
            jax.ShapeDtypeStruct((_N, _D), jnp.float32),
        ],
    )(sums, inv, xr, W2l, W3, b3.reshape(1, 1))


def kernel(x, edge_index, W1l, b1l, W1r, W2l, b2l, W2r, W3, b3):
    ei = edge_index.astype(jnp.int32)
    edges = jnp.pad(ei, ((0, 0), (0, (_EROWS_PAD - _EROWS) * _B)))
    edges = edges.reshape(2, _EROWS_PAD, _B)

    x_p = jnp.concatenate([x, jnp.zeros((_NP - _N, _D), jnp.float32)])
    sums1, cnt = _sc_agg_l1(x_p, edges)
    xr1 = _tc_pre(x_p, W1r, b1l)
    h1, inv = _tc_post1(sums1, cnt, xr1, W1l)

    [sums2] = _sc_agg_l2(h1, edges)
    xr2 = _tc_pre(h1, W2r, b2l)
    score, emb = _tc_post2(sums2, inv, xr2, W2l, W3, b3)
    return (score, emb)
